# baseline (speedup 1.0000x reference)
import functools

import jax
import jax.numpy as jnp
from jax import lax
from jax.experimental import pallas as pl
from jax.experimental.pallas import tpu as pltpu
from jax.experimental.pallas import tpu_sc as plsc

BATCH = 16384
EMB_DIM = 32
NUM_WORKERS = 32
B_PER_W = BATCH // NUM_WORKERS
IDX_CHUNK = 128
N_CHUNKS = B_PER_W // IDX_CHUNK
GROUPS = B_PER_W // 16
N_USERS = 1000000
N_ITEMS = 100000


def _score_body(x_ref, w_ref, o_ref):
    o_ref[...] = jnp.sum(x_ref[...] * w_ref[...], axis=1)


def _scores(table, w_row, blk):
    n = table.shape[0]
    return pl.pallas_call(
        _score_body,
        grid=((n + blk - 1) // blk,),
        in_specs=[
            pl.BlockSpec((blk, EMB_DIM), lambda i: (i, 0)),
            pl.BlockSpec((1, EMB_DIM), lambda i: (0, 0)),
        ],
        out_specs=pl.BlockSpec((blk,), lambda i: (i,)),
        out_shape=jax.ShapeDtypeStruct((n,), jnp.float32),
    )(table, w_row)


def _gather_body(uid_hbm, iid_hbm, us_hbm, is_hbm, bias_hbm, out_hbm,
                 uidx_v, iidx_v, uval_v, ival_v, bias_v, out_v, sem):
    cid = lax.axis_index("c")
    sid = lax.axis_index("s")
    wid = sid * 2 + cid
    base = wid * B_PER_W

    pltpu.sync_copy(bias_hbm, bias_v)
    for c in range(N_CHUNKS):
        pltpu.sync_copy(uid_hbm.at[pl.ds(base + c * IDX_CHUNK, IDX_CHUNK)],
                        uidx_v.at[c])
        pltpu.sync_copy(iid_hbm.at[pl.ds(base + c * IDX_CHUNK, IDX_CHUNK)],
                        iidx_v.at[c])
    copies = []
    for c in range(N_CHUNKS):
        copies.append(pltpu.async_copy(
            us_hbm.at[uidx_v.at[c]],
            uval_v.at[pl.ds(c * IDX_CHUNK, IDX_CHUNK)], sem))
        copies.append(pltpu.async_copy(
            is_hbm.at[iidx_v.at[c]],
            ival_v.at[pl.ds(c * IDX_CHUNK, IDX_CHUNK)], sem))
    for cp in copies:
        cp.wait()

    bias = bias_v[...]

    def group_body(g, _):
        u = uval_v[pl.ds(g * 16, 16)]
        i = ival_v[pl.ds(g * 16, 16)]
        out_v[pl.ds(g * 16, 16)] = u + i + bias
        return 0

    lax.fori_loop(0, GROUPS, group_body, 0)
    pltpu.sync_copy(out_v, out_hbm.at[pl.ds(base, B_PER_W)])


def _gather_scores(user_ids, item_ids, uscore, iscore, bias16):
    mesh = plsc.VectorSubcoreMesh(core_axis_name="c", subcore_axis_name="s")
    kern = functools.partial(
        pl.kernel,
        mesh=mesh,
        compiler_params=pltpu.CompilerParams(
            needs_layout_passes=False, disable_bounds_checks=True,
            skip_device_barrier=True),
        out_type=jax.ShapeDtypeStruct((BATCH,), jnp.float32),
        scratch_types=[
            pltpu.VMEM((N_CHUNKS, IDX_CHUNK), jnp.int32),
            pltpu.VMEM((N_CHUNKS, IDX_CHUNK), jnp.int32),
            pltpu.VMEM((B_PER_W,), jnp.float32),
            pltpu.VMEM((B_PER_W,), jnp.float32),
            pltpu.VMEM((16,), jnp.float32),
            pltpu.VMEM((B_PER_W,), jnp.float32),
            pltpu.SemaphoreType.DMA,
        ],
    )(_gather_body)
    return kern(user_ids, item_ids, uscore, iscore, bias16)


@jax.jit
def _ncf(user_ids, item_ids, user_table, item_table, W, b):
    wu = W[:EMB_DIM, 0].reshape(1, EMB_DIM)
    wi = W[EMB_DIM:, 0].reshape(1, EMB_DIM)
    uscore = _scores(user_table, wu, 8192)
    iscore = _scores(item_table, wi, 8192)
    bias16 = jnp.broadcast_to(b, (16,))
    out = _gather_scores(user_ids.astype(jnp.int32),
                         item_ids.astype(jnp.int32),
                         uscore, iscore, bias16)
    return out.reshape(BATCH, 1)


def kernel(user_ids, item_ids, user_table, item_table, W, b):
    return _ncf(user_ids, item_ids, user_table, item_table, W, b)


# speedup vs baseline: 1.0002x; 1.0002x over previous
import functools

import jax
import jax.numpy as jnp
from jax import lax
from jax.experimental import pallas as pl
from jax.experimental.pallas import tpu as pltpu
from jax.experimental.pallas import tpu_sc as plsc

BATCH = 16384
EMB_DIM = 32
NUM_WORKERS = 32
B_PER_W = BATCH // NUM_WORKERS
IDX_CHUNK = 128
N_CHUNKS = B_PER_W // IDX_CHUNK
GROUPS = B_PER_W // 16
N_USERS = 1000000
N_ITEMS = 100000


def _score_body(x_ref, w_ref, o_ref):
    o_ref[...] = jax.lax.dot_general(
        x_ref[...], w_ref[...],
        dimension_numbers=(((1,), (1,)), ((), ())),
        preferred_element_type=jnp.float32,
    ).reshape(o_ref.shape)


def _scores(table, w_row, blk):
    n = table.shape[0]
    return pl.pallas_call(
        _score_body,
        grid=((n + blk - 1) // blk,),
        in_specs=[
            pl.BlockSpec((blk, EMB_DIM), lambda i: (i, 0)),
            pl.BlockSpec((1, EMB_DIM), lambda i: (0, 0)),
        ],
        out_specs=pl.BlockSpec((blk,), lambda i: (i,)),
        out_shape=jax.ShapeDtypeStruct((n,), jnp.float32),
    )(table, w_row)


def _gather_body(uid_hbm, iid_hbm, us_hbm, is_hbm, bias_hbm, out_hbm,
                 uidx_v, iidx_v, uval_v, ival_v, bias_v, out_v, sem):
    cid = lax.axis_index("c")
    sid = lax.axis_index("s")
    wid = sid * 2 + cid
    base = wid * B_PER_W

    pltpu.sync_copy(bias_hbm, bias_v)
    for c in range(N_CHUNKS):
        pltpu.sync_copy(uid_hbm.at[pl.ds(base + c * IDX_CHUNK, IDX_CHUNK)],
                        uidx_v.at[c])
        pltpu.sync_copy(iid_hbm.at[pl.ds(base + c * IDX_CHUNK, IDX_CHUNK)],
                        iidx_v.at[c])
    copies = []
    for c in range(N_CHUNKS):
        copies.append(pltpu.async_copy(
            us_hbm.at[uidx_v.at[c]],
            uval_v.at[pl.ds(c * IDX_CHUNK, IDX_CHUNK)], sem))
        copies.append(pltpu.async_copy(
            is_hbm.at[iidx_v.at[c]],
            ival_v.at[pl.ds(c * IDX_CHUNK, IDX_CHUNK)], sem))
    for cp in copies:
        cp.wait()

    bias = bias_v[...]

    def group_body(g, _):
        u = uval_v[pl.ds(g * 16, 16)]
        i = ival_v[pl.ds(g * 16, 16)]
        out_v[pl.ds(g * 16, 16)] = u + i + bias
        return 0

    lax.fori_loop(0, GROUPS, group_body, 0)
    pltpu.sync_copy(out_v, out_hbm.at[pl.ds(base, B_PER_W)])


def _gather_scores(user_ids, item_ids, uscore, iscore, bias16):
    mesh = plsc.VectorSubcoreMesh(core_axis_name="c", subcore_axis_name="s")
    kern = functools.partial(
        pl.kernel,
        mesh=mesh,
        compiler_params=pltpu.CompilerParams(
            needs_layout_passes=False, disable_bounds_checks=True,
            skip_device_barrier=True),
        out_type=jax.ShapeDtypeStruct((BATCH,), jnp.float32),
        scratch_types=[
            pltpu.VMEM((N_CHUNKS, IDX_CHUNK), jnp.int32),
            pltpu.VMEM((N_CHUNKS, IDX_CHUNK), jnp.int32),
            pltpu.VMEM((B_PER_W,), jnp.float32),
            pltpu.VMEM((B_PER_W,), jnp.float32),
            pltpu.VMEM((16,), jnp.float32),
            pltpu.VMEM((B_PER_W,), jnp.float32),
            pltpu.SemaphoreType.DMA,
        ],
    )(_gather_body)
    return kern(user_ids, item_ids, uscore, iscore, bias16)


@jax.jit
def _ncf(user_ids, item_ids, user_table, item_table, W, b):
    wu = W[:EMB_DIM, 0].reshape(1, EMB_DIM)
    wi = W[EMB_DIM:, 0].reshape(1, EMB_DIM)
    uscore = _scores(user_table, wu, 8192)
    iscore = _scores(item_table, wi, 8192)
    bias16 = jnp.broadcast_to(b, (16,))
    out = _gather_scores(user_ids.astype(jnp.int32),
                         item_ids.astype(jnp.int32),
                         uscore, iscore, bias16)
    return out.reshape(BATCH, 1)


def kernel(user_ids, item_ids, user_table, item_table, W, b):
    return _ncf(user_ids, item_ids, user_table, item_table, W, b)
